# baseline (speedup 1.0000x reference)
import functools

import jax
import jax.numpy as jnp
from jax import lax
from jax.experimental import pallas as pl
from jax.experimental.pallas import tpu as pltpu
from jax.experimental.pallas import tpu_sc as plsc

NUM_RANKS = 13
NUM_SUITS = 4
NUM_ENH = 9
NUM_ED = 5
D = 128
LANES = 16
NCARD = NUM_RANKS * NUM_SUITS
NENHED = NUM_ENH * NUM_ED

L_SEQ = 50
R_BLOCK = 4
T_BLOCK = R_BLOCK * L_SEQ
N_GROUPS = -(-T_BLOCK // LANES)
T_PAD = N_GROUPS * LANES
N_WORKERS = 32


def _sc_body(cards_hbm, enh_hbm, ed_hbm, mask_hbm,
             rank_hbm, suit_hbm, enhe_hbm, ede_hbm, out_hbm,
             rank_v, suit_v, enhe_v, ede_v,
             card_tab, enhed_tab,
             ic0, ie0, id0, im0, ic1, ie1, id1, im1,
             stage0, stage1, sem0, sem1, isem0, isem1):
    n_rows = out_hbm.shape[0]
    nc = 2
    ns = 16
    wid = lax.axis_index("s") * nc + lax.axis_index("c")
    rows_per_w = n_rows // (nc * ns)
    n_blk = rows_per_w // R_BLOCK
    wrow = wid * rows_per_w

    pltpu.sync_copy(rank_hbm, rank_v)
    pltpu.sync_copy(suit_hbm, suit_v)
    pltpu.sync_copy(enhe_hbm, enhe_v)
    pltpu.sync_copy(ede_hbm, ede_v)

    def build_card(r, s):
        base = (s * NUM_RANKS + r) * D
        for j in range(D // LANES):
            card_tab[pl.ds(base + j * LANES, LANES)] = (
                rank_v[pl.ds(r * D + j * LANES, LANES)]
                + suit_v[pl.ds(s * D + j * LANES, LANES)])

    for s in range(NUM_SUITS):
        lax.fori_loop(0, NUM_RANKS, lambda r, _, s=s: (build_card(r, s), 0)[1], 0)

    def build_enhed(e, d):
        base = (e * NUM_ED + d) * D
        for j in range(D // LANES):
            enhed_tab[pl.ds(base + j * LANES, LANES)] = (
                enhe_v[pl.ds(e * D + j * LANES, LANES)]
                + ede_v[pl.ds(d * D + j * LANES, LANES)])

    for d in range(NUM_ED):
        lax.fori_loop(0, NUM_ENH, lambda e, _, d=d: (build_enhed(e, d), 0)[1], 0)

    zeros = jnp.zeros((LANES,), jnp.float32)
    for j in range(D // LANES):
        card_tab[pl.ds(NCARD * D + j * LANES, LANES)] = zeros
        enhed_tab[pl.ds(NENHED * D + j * LANES, LANES)] = zeros

    def fetch_idx(blk, idx_v, isem):
        base = (wrow + blk * R_BLOCK) * L_SEQ
        tsl = pl.ds(base, T_BLOCK)
        dsl = pl.ds(0, T_BLOCK)
        pltpu.async_copy(cards_hbm.at[tsl], idx_v[0].at[dsl], isem)
        pltpu.async_copy(enh_hbm.at[tsl], idx_v[1].at[dsl], isem)
        pltpu.async_copy(ed_hbm.at[tsl], idx_v[2].at[dsl], isem)
        pltpu.async_copy(mask_hbm.at[tsl], idx_v[3].at[dsl], isem)

    def wait_idx(idx_v, isem):
        for r in range(4):
            pltpu.make_async_copy(
                cards_hbm.at[pl.ds(0, T_BLOCK)], idx_v[r].at[pl.ds(0, T_BLOCK)],
                isem).wait()

    def compute_block(idx_v, stg):
        def group(g, _):
            t0 = g * LANES
            sl16 = pl.ds(t0, LANES)
            c = idx_v[0][sl16]
            e = idx_v[1][sl16]
            d = idx_v[2][sl16]
            valid = idx_v[3][sl16] > 0
            ocs = jnp.clip(jnp.where(valid, c, NCARD), 0, NCARD) * D
            oes = jnp.clip(jnp.where(valid, e * NUM_ED + d, NENHED),
                           0, NENHED) * D
            for k in range(LANES):
                oc = ocs[k]
                oe = oes[k]
                t = t0 + k
                rr = t // L_SEQ
                ll = t - rr * L_SEQ
                cvals = [card_tab[pl.ds(oc + j * LANES, LANES)]
                         for j in range(D // LANES)]
                evals = [enhed_tab[pl.ds(oe + j * LANES, LANES)]
                         for j in range(D // LANES)]
                for j in range(D // LANES):
                    stg[rr, ll, pl.ds(j * LANES, LANES)] = cvals[j] + evals[j]
            return 0

        lax.fori_loop(0, N_GROUPS, group, 0)

    def put_out(blk, stg, sem):
        rowbase = wrow + blk * R_BLOCK
        for r in range(R_BLOCK):
            pltpu.async_copy(stg.at[r], out_hbm.at[rowbase + r], sem)

    def wait_out(stg, sem):
        for r in range(R_BLOCK):
            pltpu.make_async_copy(stg.at[0], out_hbm.at[0], sem).wait()

    bufs = ((stage0, sem0, (ic0, ie0, id0, im0), isem0),
            (stage1, sem1, (ic1, ie1, id1, im1), isem1))

    fetch_idx(0, bufs[0][2], isem0)
    fetch_idx(1, bufs[1][2], isem1)

    def do_pair(i, _):
        for b, (stg, sem, idx_v, isem) in enumerate(bufs):
            blk = i * 2 + b

            @pl.when(i >= 1)
            def _wait_out():
                wait_out(stg, sem)

            wait_idx(idx_v, isem)
            compute_block(idx_v, stg)
            put_out(blk, stg, sem)

            @pl.when(blk + 2 < n_blk)
            def _prefetch():
                fetch_idx(blk + 2, idx_v, isem)
        return 0

    lax.fori_loop(0, n_blk // 2, do_pair, 0)
    for stg, sem, _, _ in bufs:
        wait_out(stg, sem)


def _card_embed(n_rows, interpret=False):
    mesh = plsc.VectorSubcoreMesh(core_axis_name="c", subcore_axis_name="s",
                                  num_cores=2, num_subcores=16)
    f = functools.partial(
        pl.kernel,
        out_type=jax.ShapeDtypeStruct((n_rows, L_SEQ, D), jnp.float32),
        mesh=mesh,
        scratch_types=[
            pltpu.VMEM((NUM_RANKS * D,), jnp.float32),
            pltpu.VMEM((NUM_SUITS * D,), jnp.float32),
            pltpu.VMEM((NUM_ENH * D,), jnp.float32),
            pltpu.VMEM((NUM_ED * D,), jnp.float32),
            pltpu.VMEM(((NCARD + 1) * D,), jnp.float32),
            pltpu.VMEM(((NENHED + 1) * D,), jnp.float32),
            pltpu.VMEM((T_PAD,), jnp.int32),
            pltpu.VMEM((T_PAD,), jnp.int32),
            pltpu.VMEM((T_PAD,), jnp.int32),
            pltpu.VMEM((T_PAD,), jnp.int32),
            pltpu.VMEM((T_PAD,), jnp.int32),
            pltpu.VMEM((T_PAD,), jnp.int32),
            pltpu.VMEM((T_PAD,), jnp.int32),
            pltpu.VMEM((T_PAD,), jnp.int32),
            pltpu.VMEM((R_BLOCK + 1, L_SEQ, D), jnp.float32),
            pltpu.VMEM((R_BLOCK + 1, L_SEQ, D), jnp.float32),
            pltpu.SemaphoreType.DMA,
            pltpu.SemaphoreType.DMA,
            pltpu.SemaphoreType.DMA,
            pltpu.SemaphoreType.DMA,
        ],
        compiler_params=pltpu.CompilerParams(use_tc_tiling_on_sc=True, needs_layout_passes=True),
        interpret=interpret,
    )
    return f(_sc_body)


def kernel(card_ids, enhancements, editions, slot_mask,
           rank_emb, suit_emb, enhancement_emb, edition_emb):
    b, l = card_ids.shape
    n_tok = b * l
    cards = card_ids.astype(jnp.int32).reshape(n_tok)
    enh = enhancements.astype(jnp.int32).reshape(n_tok)
    ed = editions.astype(jnp.int32).reshape(n_tok)
    mask = slot_mask.astype(jnp.int32).reshape(n_tok)
    toks = _card_embed(b)(
        cards, enh, ed, mask,
        rank_emb.reshape(-1), suit_emb.reshape(-1),
        enhancement_emb.reshape(-1), edition_emb.reshape(-1))
    return toks, slot_mask.astype(bool)


# speedup vs baseline: 1.0228x; 1.0228x over previous
import functools

import jax
import jax.numpy as jnp
from jax import lax
from jax.experimental import pallas as pl
from jax.experimental.pallas import tpu as pltpu
from jax.experimental.pallas import tpu_sc as plsc

NUM_RANKS = 13
NUM_SUITS = 4
NUM_ENH = 9
NUM_ED = 5
D = 128
LANES = 16
PK = 2 * LANES
NCARD = NUM_RANKS * NUM_SUITS
NENHED = NUM_ENH * NUM_ED

L_SEQ = 50
R_BLOCK = 8
T_BLOCK = R_BLOCK * L_SEQ
N_WORKERS = 32


def _sc_body(cards_hbm, enh_hbm, ed_hbm, mask_hbm,
             rank_hbm, suit_hbm, enhe_hbm, ede_hbm, out_hbm,
             rank_v, suit_v, enhe_v, ede_v,
             card_tab, enhed_tab,
             ic0, ie0, id0, im0, ic1, ie1, id1, im1,
             stage0, stage1, sem0, sem1, isem0, isem1):
    n_rows = out_hbm.shape[0]
    nc = 2
    ns = 16
    wid = lax.axis_index("s") * nc + lax.axis_index("c")
    rows_per_w = n_rows // (nc * ns)
    n_blk = rows_per_w // R_BLOCK
    wrow = wid * rows_per_w

    pltpu.sync_copy(rank_hbm, rank_v)
    pltpu.sync_copy(suit_hbm, suit_v)
    pltpu.sync_copy(enhe_hbm, enhe_v)
    pltpu.sync_copy(ede_hbm, ede_v)

    def build_card(r, s):
        row = s * NUM_RANKS + r
        for j in range(D // LANES):
            sl = pl.ds(j * LANES, LANES)
            card_tab[row, sl] = rank_v[r, sl] + suit_v[s, sl]

    for s in range(NUM_SUITS):
        lax.fori_loop(0, NUM_RANKS, lambda r, _, s=s: (build_card(r, s), 0)[1], 0)

    def build_enhed(e, d):
        row = e * NUM_ED + d
        for j in range(D // LANES):
            sl = pl.ds(j * LANES, LANES)
            enhed_tab[row, sl] = enhe_v[e, sl] + ede_v[d, sl]

    for d in range(NUM_ED):
        lax.fori_loop(0, NUM_ENH, lambda e, _, d=d: (build_enhed(e, d), 0)[1], 0)

    zeros = jnp.zeros((LANES,), jnp.float32)
    for j in range(D // LANES):
        sl = pl.ds(j * LANES, LANES)
        card_tab[NCARD, sl] = zeros
        enhed_tab[NENHED, sl] = zeros

    def fetch_idx(blk, idx_v, isem):
        base = (wrow + blk * R_BLOCK) * L_SEQ
        tsl = pl.ds(base, T_BLOCK)
        pltpu.async_copy(cards_hbm.at[tsl], idx_v[0], isem)
        pltpu.async_copy(enh_hbm.at[tsl], idx_v[1], isem)
        pltpu.async_copy(ed_hbm.at[tsl], idx_v[2], isem)
        pltpu.async_copy(mask_hbm.at[tsl], idx_v[3], isem)

    def wait_idx(idx_v, isem):
        for r in range(4):
            pltpu.make_async_copy(
                cards_hbm.at[pl.ds(0, T_BLOCK)], idx_v[r], isem).wait()

    def compute_block(idx_v, stg):
        def group(g, _):
            sl16 = pl.ds(g * LANES, LANES)
            c = idx_v[0][sl16]
            e = idx_v[1][sl16]
            d = idx_v[2][sl16]
            valid = idx_v[3][sl16] > 0
            ocs = jnp.where(valid, c, NCARD)
            oes = jnp.where(valid, e * NUM_ED + d, NENHED)
            for k in range(LANES):
                rc = ocs[k]
                re = oes[k]
                t = g * LANES + k
                cvals = [card_tab[rc, pl.ds(j * LANES, LANES)]
                         for j in range(D // LANES)]
                evals = [enhed_tab[re, pl.ds(j * LANES, LANES)]
                         for j in range(D // LANES)]
                for j in range(D // LANES):
                    stg[t, pl.ds(j * LANES, LANES)] = cvals[j] + evals[j]
            return 0

        lax.fori_loop(0, T_BLOCK // LANES, group, 0)

    def put_out(blk, stg, sem):
        rowbase = wrow + blk * R_BLOCK
        for r in range(R_BLOCK):
            pltpu.async_copy(stg.at[pl.ds(r * L_SEQ, L_SEQ)],
                             out_hbm.at[rowbase + r], sem)

    def wait_out(stg, sem):
        for r in range(R_BLOCK):
            pltpu.make_async_copy(stg.at[pl.ds(0, L_SEQ)],
                                  out_hbm.at[0], sem).wait()

    bufs = ((stage0, sem0, (ic0, ie0, id0, im0), isem0),
            (stage1, sem1, (ic1, ie1, id1, im1), isem1))

    fetch_idx(0, bufs[0][2], isem0)
    fetch_idx(1, bufs[1][2], isem1)

    def do_pair(i, _):
        for b, (stg, sem, idx_v, isem) in enumerate(bufs):
            blk = i * 2 + b

            @pl.when(i >= 1)
            def _wait_out():
                wait_out(stg, sem)

            wait_idx(idx_v, isem)
            compute_block(idx_v, stg)
            put_out(blk, stg, sem)

            @pl.when(blk + 2 < n_blk)
            def _prefetch():
                fetch_idx(blk + 2, idx_v, isem)
        return 0

    lax.fori_loop(0, n_blk // 2, do_pair, 0)
    for stg, sem, _, _ in bufs:
        wait_out(stg, sem)


def _card_embed(n_rows, interpret=False):
    mesh = plsc.VectorSubcoreMesh(core_axis_name="c", subcore_axis_name="s",
                                  num_cores=2, num_subcores=16)
    f = functools.partial(
        pl.kernel,
        out_type=jax.ShapeDtypeStruct((n_rows, L_SEQ, D), jnp.float32),
        mesh=mesh,
        scratch_types=[
            pltpu.VMEM((NUM_RANKS, D), jnp.float32),
            pltpu.VMEM((NUM_SUITS, D), jnp.float32),
            pltpu.VMEM((NUM_ENH, D), jnp.float32),
            pltpu.VMEM((NUM_ED, D), jnp.float32),
            pltpu.VMEM((NCARD + 1, D), jnp.float32),
            pltpu.VMEM((NENHED + 1, D), jnp.float32),
            pltpu.VMEM((T_BLOCK,), jnp.int32),
            pltpu.VMEM((T_BLOCK,), jnp.int32),
            pltpu.VMEM((T_BLOCK,), jnp.int32),
            pltpu.VMEM((T_BLOCK,), jnp.int32),
            pltpu.VMEM((T_BLOCK,), jnp.int32),
            pltpu.VMEM((T_BLOCK,), jnp.int32),
            pltpu.VMEM((T_BLOCK,), jnp.int32),
            pltpu.VMEM((T_BLOCK,), jnp.int32),
            pltpu.VMEM((T_BLOCK, D), jnp.float32),
            pltpu.VMEM((T_BLOCK, D), jnp.float32),
            pltpu.SemaphoreType.DMA,
            pltpu.SemaphoreType.DMA,
            pltpu.SemaphoreType.DMA,
            pltpu.SemaphoreType.DMA,
        ],
        interpret=interpret,
    )
    return f(_sc_body)


def kernel(card_ids, enhancements, editions, slot_mask,
           rank_emb, suit_emb, enhancement_emb, edition_emb):
    b, l = card_ids.shape
    n_tok = b * l
    cards = card_ids.astype(jnp.int32).reshape(n_tok)
    enh = enhancements.astype(jnp.int32).reshape(n_tok)
    ed = editions.astype(jnp.int32).reshape(n_tok)
    mask = slot_mask.astype(jnp.int32).reshape(n_tok)
    toks = _card_embed(b)(
        cards, enh, ed, mask, rank_emb, suit_emb, enhancement_emb, edition_emb)
    return toks, slot_mask.astype(bool)
